# baseline (speedup 1.0000x reference)
import functools
import jax
import jax.numpy as jnp
from jax import lax
from jax.experimental import pallas as pl
from jax.experimental.pallas import tpu as pltpu
from jax.experimental.pallas import tpu_sc as plsc

N = 10000
E = 160000
D_IN = 256
HID = 256
N_CLASSES = 40
HEADS = 4

NP = 10240
TN = 1024
CP2 = 128

NT = 32
BS = NP // NT
EP = E + N
EPP = 172032
CH = 8192
FB = 4096
CAP = 176128
CHW = 2048


def _l1_mm_kernel(x_ref, w_ref, u_ref, h_ref, a_ref):
    xb = x_ref[...]
    for g in range(HEADS):
        h_ref[g] = jnp.dot(xb, w_ref[g], preferred_element_type=jnp.float32)
    a = jnp.dot(xb, u_ref[...], preferred_element_type=jnp.float32)
    a_ref[...] = a.T


def _l1_matmul(xp, Wr, U):
    grid = (NP // TN,)
    return pl.pallas_call(
        _l1_mm_kernel,
        grid=grid,
        in_specs=[
            pl.BlockSpec((TN, D_IN), lambda i: (i, 0)),
            pl.BlockSpec((HEADS, D_IN, HID), lambda i: (0, 0, 0)),
            pl.BlockSpec((D_IN, 8), lambda i: (0, 0)),
        ],
        out_specs=[
            pl.BlockSpec((HEADS, TN, HID), lambda i: (0, i, 0)),
            pl.BlockSpec((8, TN), lambda i: (0, i)),
        ],
        out_shape=[
            jax.ShapeDtypeStruct((HEADS, NP, HID), jnp.float32),
            jax.ShapeDtypeStruct((8, NP), jnp.float32),
        ],
    )(xp, Wr, U)


def _l2_mm_kernel(o_ref, b_ref, w_ref, u_ref, h_ref, a_ref):
    m = (o_ref[0] + o_ref[1] + o_ref[2] + o_ref[3]) * 0.25 + b_ref[...]
    hp = jnp.maximum(m, 0.0)
    for g in range(HEADS):
        h_ref[g] = jnp.dot(hp, w_ref[g], preferred_element_type=jnp.float32)
    a = jnp.dot(hp, u_ref[...], preferred_element_type=jnp.float32)
    a_ref[...] = a.T


def _l2_matmul(out1, b1, W2r, U2):
    grid = (NP // TN,)
    return pl.pallas_call(
        _l2_mm_kernel,
        grid=grid,
        in_specs=[
            pl.BlockSpec((HEADS, TN, HID), lambda i: (0, i, 0)),
            pl.BlockSpec((HID,), lambda i: (0,)),
            pl.BlockSpec((HEADS, HID, CP2), lambda i: (0, 0, 0)),
            pl.BlockSpec((HID, 8), lambda i: (0, 0)),
        ],
        out_specs=[
            pl.BlockSpec((HEADS, TN, CP2), lambda i: (0, i, 0)),
            pl.BlockSpec((8, TN), lambda i: (0, i)),
        ],
        out_shape=[
            jax.ShapeDtypeStruct((HEADS, NP, CP2), jnp.float32),
            jax.ShapeDtypeStruct((8, NP), jnp.float32),
        ],
    )(out1, b1, W2r, U2)


def _final_kernel(o_ref, b_ref, out_ref):
    out_ref[...] = (o_ref[0] + o_ref[1] + o_ref[2] + o_ref[3]) * 0.25 + b_ref[...]


def _final(out2, b2p):
    grid = (NP // TN,)
    return pl.pallas_call(
        _final_kernel,
        grid=grid,
        in_specs=[
            pl.BlockSpec((HEADS, TN, CP2), lambda i: (0, i, 0)),
            pl.BlockSpec((CP2,), lambda i: (0,)),
        ],
        out_specs=pl.BlockSpec((TN, CP2), lambda i: (i, 0)),
        out_shape=jax.ShapeDtypeStruct((NP, CP2), jnp.float32),
    )(out2, b2p)


def _sc_mesh():
    return plsc.VectorSubcoreMesh(core_axis_name="c", subcore_axis_name="s")


def _wid():
    return lax.axis_index("s") * 2 + lax.axis_index("c")


def _bucketize(src, dst):
    @functools.partial(
        pl.kernel,
        out_type=[
            jax.ShapeDtypeStruct((NT, CAP), jnp.int32),
            jax.ShapeDtypeStruct((NT * 128,), jnp.int32),
        ],
        mesh=_sc_mesh(),
        compiler_params=pltpu.CompilerParams(needs_layout_passes=False),
        scratch_types=[
            pltpu.VMEM((CH,), jnp.int32),
            pltpu.VMEM((CH,), jnp.int32),
            pltpu.VMEM((FB + 16,), jnp.int32),
            pltpu.VMEM((128,), jnp.int32),
        ],
    )
    def k(src_hbm, dst_hbm, wpack_hbm, counts_hbm, sbuf, dbuf, cbuf, cntv):
        t = _wid()
        base = t * BS
        wrow = wpack_hbm.at[t]

        def chunk_body(ci, carry):
            off = pl.multiple_of(ci * CH, 128)
            pltpu.sync_copy(src_hbm.at[pl.ds(off, CH)], sbuf)
            pltpu.sync_copy(dst_hbm.at[pl.ds(off, CH)], dbuf)

            def group_body(gi, c):
                cnt, gb = c
                s16 = sbuf[pl.ds(gi * 16, 16)]
                d16 = dbuf[pl.ds(gi * 16, 16)]
                dl = d16 - base
                inb = (dl >= 0) & (dl < BS)
                pk = (s16 << 9) | (d16 - base)
                key = 1 - inb.astype(jnp.int32)
                _, pksorted = plsc.sort_key_val(key, pk)
                cbuf[pl.ds(cnt, 16)] = pksorted
                cnt = cnt + jnp.sum(inb.astype(jnp.int32))

                def flush(c2):
                    cnt2, gb2 = c2
                    gba = pl.multiple_of(gb2, 128)
                    pltpu.sync_copy(cbuf.at[pl.ds(0, FB)],
                                    wrow.at[pl.ds(gba, FB)])
                    cbuf[pl.ds(0, 16)] = cbuf[pl.ds(FB, 16)]
                    return cnt2 - FB, gb2 + FB

                return lax.cond(cnt >= FB, flush, lambda c2: c2, (cnt, gb))

            return lax.fori_loop(0, CH // 16, group_body, carry)

        cnt, gb = lax.fori_loop(0, EPP // CH, chunk_body,
                                (jnp.int32(0), jnp.int32(0)))
        gba = pl.multiple_of(gb, 128)
        pltpu.sync_copy(cbuf.at[pl.ds(0, FB)], wrow.at[pl.ds(gba, FB)])
        total = jnp.full((16,), gb + cnt, jnp.int32)
        for i in range(8):
            cntv[pl.ds(i * 16, 16)] = total
        coff = pl.multiple_of(t * 128, 128)
        pltpu.sync_copy(cntv, counts_hbm.at[pl.ds(coff, 128)])

    return k(src, dst)


def _make_agg(CV, CVA=None):
    nv = (CVA or CV) // 16

    @functools.partial(
        pl.kernel,
        out_type=jax.ShapeDtypeStruct((HEADS, NP * CV), jnp.float32),
        mesh=_sc_mesh(),
        compiler_params=pltpu.CompilerParams(needs_layout_passes=False),
        scratch_types=[
            pltpu.VMEM((NP,), jnp.float32),
            pltpu.VMEM((NP,), jnp.float32),
            pltpu.VMEM((BS,), jnp.float32),
            pltpu.VMEM((BS * CV,), jnp.float32),
            pltpu.VMEM((CHW,), jnp.int32),
            pltpu.VMEM((16 * CV,), jnp.float32),
            pltpu.VMEM((16 * CV,), jnp.float32),
            pltpu.VMEM((16 * CV,), jnp.float32),
            pltpu.VMEM((16 * CV,), jnp.float32),
            pltpu.VMEM((128,), jnp.int32),
            pltpu.SemaphoreType.DMA,
            pltpu.SemaphoreType.DMA,
            pltpu.SemaphoreType.DMA,
            pltpu.SemaphoreType.DMA,
        ],
    )
    def k(h_hbm, aT_hbm, wpack_hbm, counts_hbm, out_hbm,
          asrc_tab, adst_tab, denom, outblk, wbuf,
          stag0, stag1, stag2, stag3, cntv,
          sem0, sem1, sem2, sem3):
        stags = (stag0, stag1, stag2, stag3)
        sems = (sem0, sem1, sem2, sem3)
        t = _wid()
        base = t * BS
        wrow = wpack_hbm.at[t]
        coff = pl.multiple_of(t * 128, 128)
        pltpu.sync_copy(counts_hbm.at[pl.ds(coff, 128)], cntv)
        cnt = cntv[pl.ds(0, 16)][0]
        nchunks = (cnt + (CHW - 1)) // CHW
        iota16 = lax.iota(jnp.int32, 16)
        zidx = jnp.zeros((16,), jnp.int32)

        def decode(ci, gi):
            pk = wbuf[pl.ds(gi * 16, 16)]
            valid = (ci * CHW + gi * 16 + iota16) < cnt
            srcv = jnp.where(valid, pk >> 9, 0)
            dl = jnp.where(valid, pk & (512 - 1), 0)
            return srcv, dl, valid

        def edge_ex(srcv, dl, valid):
            asv = plsc.load_gather(asrc_tab, [srcv])
            adv = plsc.load_gather(adst_tab, [dl + base])
            al = asv + adv
            al = jnp.where(al > 0, al, al * 0.2)
            return jnp.where(valid, jnp.exp(al), 0.0)

        def head_body(g, _):
            hg = h_hbm.at[g]
            pltpu.sync_copy(aT_hbm.at[g], asrc_tab)
            pltpu.sync_copy(aT_hbm.at[HEADS + g], adst_tab)

            z16 = jnp.zeros((16,), jnp.float32)

            def zden(i, _):
                denom[pl.ds(i * 16, 16)] = z16
                return 0
            lax.fori_loop(0, BS // 16, zden, 0)

            def zout(r, _):
                outblk[pl.ds(r * 16, 16)] = z16
                return 0
            lax.fori_loop(0, BS * CV // 16, zout, 0)

            def p1_chunk(ci, _):
                woff = pl.multiple_of(ci * CHW, 128)
                pltpu.sync_copy(wrow.at[pl.ds(woff, CHW)], wbuf)
                ng = jnp.minimum(CHW // 16, (cnt - ci * CHW + 15) // 16)

                def p1_group(gi, _):
                    srcv, dl, valid = decode(ci, gi)
                    ex = edge_ex(srcv, dl, valid)
                    for i in range(16):
                        plsc.addupdate_scatter(denom, [dl], ex,
                                               mask=iota16 == i)
                    return 0
                return lax.fori_loop(0, ng, p1_group, 0)
            lax.fori_loop(0, nchunks, p1_chunk, 0)

            def process(ci, gi, stag):
                srcv, dl, valid = decode(ci, gi)
                ex = edge_ex(srcv, dl, valid)
                dload = plsc.load_gather(denom, [dl])
                coefv = ex / (dload + 1e-16)
                cev_l = [jnp.full((16,), coefv[e], jnp.float32)
                         for e in range(16)]
                ro_l = [dl[e] * CV for e in range(16)]
                for v in range(nv):
                    svals = [cev_l[e] * stag[pl.ds(e * CV + v * 16, 16)]
                             for e in range(16)]
                    for e in range(16):
                        plsc.addupdate(outblk.at[pl.ds(ro_l[e] + v * 16, 16)],
                                       svals[e])
                return 0

            def p2_chunk(ci, _):
                woff = pl.multiple_of(ci * CHW, 128)
                pltpu.sync_copy(wrow.at[pl.ds(woff, CHW)], wbuf)
                ng = jnp.minimum(CHW // 16, (cnt - ci * CHW + 15) // 16)

                def dec(gi):
                    srcv, _, _ = decode(ci, gi)
                    return srcv

                drain_src = out_hbm.at[0, pl.ds(0, 16 * CV)]

                def issue_g(gi, b):
                    gi_c = jnp.minimum(gi, ng - 1)
                    srcv = dec(gi_c)
                    for e in range(16):
                        pltpu.async_copy(hg.at[srcv[e]],
                                         stags[b].at[pl.ds(e * CV, CV)],
                                         sems[b])

                def wait_g(b):
                    pltpu.make_async_copy(drain_src, stags[b], sems[b]).wait()

                for b in range(4):
                    issue_g(jnp.int32(b), b)

                def body4(j, _):
                    for q in range(4):
                        g4 = j * 4 + q
                        wait_g(q)
                        process(ci, g4, stags[q])
                        issue_g(g4 + 4, q)
                    return 0

                nquads = (ng + 3) // 4
                lax.fori_loop(0, nquads, body4, 0)
                for b in range(4):
                    wait_g(b)
                return 0
            lax.fori_loop(0, nchunks, p2_chunk, 0)

            pltpu.sync_copy(
                outblk,
                out_hbm.at[g, pl.ds(pl.multiple_of(base * CV, 128), BS * CV)])
            return 0

        lax.fori_loop(0, HEADS, head_body, 0)

    return k


_AGG_HID = _make_agg(HID)
_AGG_CLS = _make_agg(CP2, CVA=48)


def kernel(x, edge_index, W1, att_src1, att_dst1, b1, W2, att_src2, att_dst2, b2):
    loop = jnp.arange(N, dtype=edge_index.dtype)
    pad = jnp.zeros(EPP - EP, dtype=edge_index.dtype)
    src = jnp.concatenate([edge_index[0], loop, pad]).astype(jnp.int32)
    dst = jnp.concatenate([edge_index[1], loop, pad + NP]).astype(jnp.int32)

    xp = jnp.pad(x, ((0, NP - N), (0, 0)))
    W1r = W1.reshape(D_IN, HEADS, HID).transpose(1, 0, 2)
    U1 = jnp.concatenate([
        jnp.einsum('hdc,hc->dh', W1r, att_src1),
        jnp.einsum('hdc,hc->dh', W1r, att_dst1),
    ], axis=1)

    W2r = W2.reshape(HID, HEADS, N_CLASSES).transpose(1, 0, 2)
    W2p = jnp.pad(W2r, ((0, 0), (0, 0), (0, CP2 - N_CLASSES)))
    U2 = jnp.concatenate([
        jnp.einsum('hdc,hc->dh', W2r, att_src2),
        jnp.einsum('hdc,hc->dh', W2r, att_dst2),
    ], axis=1)
    b2p = jnp.pad(b2, (0, CP2 - N_CLASSES))

    wpack, counts = _bucketize(src, dst)
    h1, aT1 = _l1_matmul(xp, W1r, U1)
    out1 = _AGG_HID(h1, aT1, wpack, counts).reshape(HEADS, NP, HID)
    h2, aT2 = _l2_matmul(out1, b1, W2p, U2)
    out2 = _AGG_CLS(h2, aT2, wpack, counts).reshape(HEADS, NP, CP2)
    res = _final(out2, b2p)
    return res[:N, :N_CLASSES]


# speedup vs baseline: 1.2169x; 1.2169x over previous
import functools
import jax
import jax.numpy as jnp
from jax import lax
from jax.experimental import pallas as pl
from jax.experimental.pallas import tpu as pltpu
from jax.experimental.pallas import tpu_sc as plsc

N = 10000
E = 160000
D_IN = 256
HID = 256
N_CLASSES = 40
HEADS = 4

NP = 10240
TN = 1024
CP2 = 128

NT = 32
BS = NP // NT
EP = E + N
EPP = 172032
CH = 8192
FB = 4096
CAP = 176128
CHW = 2048


def _l1_mm_kernel(x_ref, w_ref, u_ref, h_ref, a_ref):
    xb = x_ref[...]
    for g in range(HEADS):
        h_ref[g] = jnp.dot(xb, w_ref[g], preferred_element_type=jnp.float32)
    a = jnp.dot(xb, u_ref[...], preferred_element_type=jnp.float32)
    a_ref[...] = a.T


def _l1_matmul(xp, Wr, U):
    grid = (NP // TN,)
    return pl.pallas_call(
        _l1_mm_kernel,
        grid=grid,
        in_specs=[
            pl.BlockSpec((TN, D_IN), lambda i: (i, 0)),
            pl.BlockSpec((HEADS, D_IN, HID), lambda i: (0, 0, 0)),
            pl.BlockSpec((D_IN, 8), lambda i: (0, 0)),
        ],
        out_specs=[
            pl.BlockSpec((HEADS, TN, HID), lambda i: (0, i, 0)),
            pl.BlockSpec((8, TN), lambda i: (0, i)),
        ],
        out_shape=[
            jax.ShapeDtypeStruct((HEADS, NP, HID), jnp.float32),
            jax.ShapeDtypeStruct((8, NP), jnp.float32),
        ],
    )(xp, Wr, U)


def _l2_mm_kernel(o_ref, b_ref, w_ref, u_ref, h_ref, a_ref):
    m = (o_ref[0] + o_ref[1] + o_ref[2] + o_ref[3]) * 0.25 + b_ref[...]
    hp = jnp.maximum(m, 0.0)
    for g in range(HEADS):
        h_ref[g] = jnp.dot(hp, w_ref[g], preferred_element_type=jnp.float32)
    a = jnp.dot(hp, u_ref[...], preferred_element_type=jnp.float32)
    a_ref[...] = a.T


def _l2_matmul(out1, b1, W2r, U2):
    grid = (NP // TN,)
    return pl.pallas_call(
        _l2_mm_kernel,
        grid=grid,
        in_specs=[
            pl.BlockSpec((HEADS, TN, HID), lambda i: (0, i, 0)),
            pl.BlockSpec((HID,), lambda i: (0,)),
            pl.BlockSpec((HEADS, HID, CP2), lambda i: (0, 0, 0)),
            pl.BlockSpec((HID, 8), lambda i: (0, 0)),
        ],
        out_specs=[
            pl.BlockSpec((HEADS, TN, CP2), lambda i: (0, i, 0)),
            pl.BlockSpec((8, TN), lambda i: (0, i)),
        ],
        out_shape=[
            jax.ShapeDtypeStruct((HEADS, NP, CP2), jnp.float32),
            jax.ShapeDtypeStruct((8, NP), jnp.float32),
        ],
    )(out1, b1, W2r, U2)


def _final_kernel(o_ref, b_ref, out_ref):
    out_ref[...] = (o_ref[0] + o_ref[1] + o_ref[2] + o_ref[3]) * 0.25 + b_ref[...]


def _final(out2, b2p):
    grid = (NP // TN,)
    return pl.pallas_call(
        _final_kernel,
        grid=grid,
        in_specs=[
            pl.BlockSpec((HEADS, TN, CP2), lambda i: (0, i, 0)),
            pl.BlockSpec((CP2,), lambda i: (0,)),
        ],
        out_specs=pl.BlockSpec((TN, CP2), lambda i: (i, 0)),
        out_shape=jax.ShapeDtypeStruct((NP, CP2), jnp.float32),
    )(out2, b2p)


def _sc_mesh():
    return plsc.VectorSubcoreMesh(core_axis_name="c", subcore_axis_name="s")


def _wid():
    return lax.axis_index("s") * 2 + lax.axis_index("c")


def _bucketize(src, dst):
    @functools.partial(
        pl.kernel,
        out_type=[
            jax.ShapeDtypeStruct((NT, CAP), jnp.int32),
            jax.ShapeDtypeStruct((NT * 128,), jnp.int32),
        ],
        mesh=_sc_mesh(),
        compiler_params=pltpu.CompilerParams(needs_layout_passes=False),
        scratch_types=[
            pltpu.VMEM((CH,), jnp.int32),
            pltpu.VMEM((CH,), jnp.int32),
            pltpu.VMEM((FB + 16,), jnp.int32),
            pltpu.VMEM((128,), jnp.int32),
        ],
    )
    def k(src_hbm, dst_hbm, wpack_hbm, counts_hbm, sbuf, dbuf, cbuf, cntv):
        t = _wid()
        base = t * BS
        wrow = wpack_hbm.at[t]

        def chunk_body(ci, carry):
            off = pl.multiple_of(ci * CH, 128)
            pltpu.sync_copy(src_hbm.at[pl.ds(off, CH)], sbuf)
            pltpu.sync_copy(dst_hbm.at[pl.ds(off, CH)], dbuf)

            def group_body(gi, c):
                cnt, gb = c
                s16 = sbuf[pl.ds(gi * 16, 16)]
                d16 = dbuf[pl.ds(gi * 16, 16)]
                dl = d16 - base
                inb = (dl >= 0) & (dl < BS)
                pk = (s16 << 9) | (d16 - base)
                key = 1 - inb.astype(jnp.int32)
                _, pksorted = plsc.sort_key_val(key, pk)
                cbuf[pl.ds(cnt, 16)] = pksorted
                cnt = cnt + jnp.sum(inb.astype(jnp.int32))

                def flush(c2):
                    cnt2, gb2 = c2
                    gba = pl.multiple_of(gb2, 128)
                    pltpu.sync_copy(cbuf.at[pl.ds(0, FB)],
                                    wrow.at[pl.ds(gba, FB)])
                    cbuf[pl.ds(0, 16)] = cbuf[pl.ds(FB, 16)]
                    return cnt2 - FB, gb2 + FB

                return lax.cond(cnt >= FB, flush, lambda c2: c2, (cnt, gb))

            return lax.fori_loop(0, CH // 16, group_body, carry)

        cnt, gb = lax.fori_loop(0, EPP // CH, chunk_body,
                                (jnp.int32(0), jnp.int32(0)))
        gba = pl.multiple_of(gb, 128)
        pltpu.sync_copy(cbuf.at[pl.ds(0, FB)], wrow.at[pl.ds(gba, FB)])
        total = jnp.full((16,), gb + cnt, jnp.int32)
        for i in range(8):
            cntv[pl.ds(i * 16, 16)] = total
        coff = pl.multiple_of(t * 128, 128)
        pltpu.sync_copy(cntv, counts_hbm.at[pl.ds(coff, 128)])

    return k(src, dst)


def _make_agg(CV, CVA=None):
    nv = (CVA or CV) // 16

    @functools.partial(
        pl.kernel,
        out_type=jax.ShapeDtypeStruct((HEADS, NP * CV), jnp.float32),
        mesh=_sc_mesh(),
        compiler_params=pltpu.CompilerParams(needs_layout_passes=False),
        scratch_types=[
            pltpu.VMEM((NP,), jnp.float32),
            pltpu.VMEM((NP,), jnp.float32),
            pltpu.VMEM((BS,), jnp.float32),
            pltpu.VMEM((BS * CV,), jnp.float32),
            pltpu.VMEM((CHW,), jnp.int32),
            pltpu.VMEM((16 * CV,), jnp.float32),
            pltpu.VMEM((16 * CV,), jnp.float32),
            pltpu.VMEM((128,), jnp.int32),
            pltpu.SemaphoreType.DMA,
            pltpu.SemaphoreType.DMA,
        ],
    )
    def k(h_hbm, aT_hbm, wpack_hbm, counts_hbm, out_hbm,
          asrc_tab, adst_tab, denom, outblk, wbuf, stagA, stagB, cntv,
          semA, semB):
        t = _wid()
        base = t * BS
        wrow = wpack_hbm.at[t]
        coff = pl.multiple_of(t * 128, 128)
        pltpu.sync_copy(counts_hbm.at[pl.ds(coff, 128)], cntv)
        cnt = cntv[pl.ds(0, 16)][0]
        nchunks = (cnt + (CHW - 1)) // CHW
        iota16 = lax.iota(jnp.int32, 16)
        zidx = jnp.zeros((16,), jnp.int32)

        def decode(ci, gi):
            pk = wbuf[pl.ds(gi * 16, 16)]
            valid = (ci * CHW + gi * 16 + iota16) < cnt
            srcv = jnp.where(valid, pk >> 9, 0)
            dl = jnp.where(valid, pk & (512 - 1), 0)
            return srcv, dl, valid

        def edge_ex(srcv, dl, valid):
            asv = plsc.load_gather(asrc_tab, [srcv])
            adv = plsc.load_gather(adst_tab, [dl + base])
            al = asv + adv
            al = jnp.where(al > 0, al, al * 0.2)
            return jnp.where(valid, jnp.exp(al), 0.0)

        def head_body(g, _):
            hg = h_hbm.at[g]
            pltpu.sync_copy(aT_hbm.at[g], asrc_tab)
            pltpu.sync_copy(aT_hbm.at[HEADS + g], adst_tab)

            z16 = jnp.zeros((16,), jnp.float32)

            def zden(i, _):
                denom[pl.ds(i * 16, 16)] = z16
                return 0
            lax.fori_loop(0, BS // 16, zden, 0)

            def zout(r, _):
                outblk[pl.ds(r * 16, 16)] = z16
                return 0
            lax.fori_loop(0, BS * CV // 16, zout, 0)

            def p1_chunk(ci, _):
                woff = pl.multiple_of(ci * CHW, 128)
                pltpu.sync_copy(wrow.at[pl.ds(woff, CHW)], wbuf)
                ng = jnp.minimum(CHW // 16, (cnt - ci * CHW + 15) // 16)

                def p1_group(gi, _):
                    srcv, dl, valid = decode(ci, gi)
                    ex = edge_ex(srcv, dl, valid)
                    for i in range(16):
                        plsc.addupdate_scatter(denom, [dl], ex,
                                               mask=iota16 == i)
                    return 0
                return lax.fori_loop(0, ng, p1_group, 0)
            lax.fori_loop(0, nchunks, p1_chunk, 0)

            def process(ci, gi, stag):
                srcv, dl, valid = decode(ci, gi)
                ex = edge_ex(srcv, dl, valid)
                dload = plsc.load_gather(denom, [dl])
                coefv = ex / (dload + 1e-16)
                cev_l = [jnp.full((16,), coefv[e], jnp.float32)
                         for e in range(16)]
                ro_l = [dl[e] * CV for e in range(16)]
                for v in range(nv):
                    svals = [cev_l[e] * stag[pl.ds(e * CV + v * 16, 16)]
                             for e in range(16)]
                    for e in range(16):
                        plsc.addupdate(outblk.at[pl.ds(ro_l[e] + v * 16, 16)],
                                       svals[e])
                return 0

            def p2_chunk(ci, _):
                woff = pl.multiple_of(ci * CHW, 128)
                pltpu.sync_copy(wrow.at[pl.ds(woff, CHW)], wbuf)
                ng = jnp.minimum(CHW // 16, (cnt - ci * CHW + 15) // 16)

                def dec(gi):
                    srcv, _, _ = decode(ci, gi)
                    return srcv

                def issue_g(gi, stag, sem):
                    gi_c = jnp.minimum(gi, ng - 1)
                    srcv = dec(gi_c)
                    for e in range(16):
                        pltpu.async_copy(hg.at[srcv[e]],
                                         stag.at[pl.ds(e * CV, CV)], sem)

                drain_src = out_hbm.at[0, pl.ds(0, 16 * CV)]

                def wait_g(stag, sem):
                    pltpu.make_async_copy(drain_src, stag, sem).wait()

                issue_g(jnp.int32(0), stagA, semA)

                def body2(j, _):
                    gA = j * 2
                    gB = j * 2 + 1
                    issue_g(gB, stagB, semB)
                    wait_g(stagA, semA)
                    process(ci, gA, stagA)
                    issue_g(gA + 2, stagA, semA)
                    wait_g(stagB, semB)
                    process(ci, gB, stagB)
                    return 0

                npairs = (ng + 1) // 2
                lax.fori_loop(0, npairs, body2, 0)
                wait_g(stagA, semA)
                return 0
            lax.fori_loop(0, nchunks, p2_chunk, 0)

            pltpu.sync_copy(
                outblk,
                out_hbm.at[g, pl.ds(pl.multiple_of(base * CV, 128), BS * CV)])
            return 0

        lax.fori_loop(0, HEADS, head_body, 0)

    return k


_AGG_HID = _make_agg(HID)
_AGG_CLS = _make_agg(CP2, CVA=48)


def kernel(x, edge_index, W1, att_src1, att_dst1, b1, W2, att_src2, att_dst2, b2):
    loop = jnp.arange(N, dtype=edge_index.dtype)
    pad = jnp.zeros(EPP - EP, dtype=edge_index.dtype)
    src = jnp.concatenate([edge_index[0], loop, pad]).astype(jnp.int32)
    dst = jnp.concatenate([edge_index[1], loop, pad + NP]).astype(jnp.int32)

    xp = jnp.pad(x, ((0, NP - N), (0, 0)))
    W1r = W1.reshape(D_IN, HEADS, HID).transpose(1, 0, 2)
    U1 = jnp.concatenate([
        jnp.einsum('hdc,hc->dh', W1r, att_src1),
        jnp.einsum('hdc,hc->dh', W1r, att_dst1),
    ], axis=1)

    W2r = W2.reshape(HID, HEADS, N_CLASSES).transpose(1, 0, 2)
    W2p = jnp.pad(W2r, ((0, 0), (0, 0), (0, CP2 - N_CLASSES)))
    U2 = jnp.concatenate([
        jnp.einsum('hdc,hc->dh', W2r, att_src2),
        jnp.einsum('hdc,hc->dh', W2r, att_dst2),
    ], axis=1)
    b2p = jnp.pad(b2, (0, CP2 - N_CLASSES))

    wpack, counts = _bucketize(src, dst)
    h1, aT1 = _l1_matmul(xp, W1r, U1)
    out1 = _AGG_HID(h1, aT1, wpack, counts).reshape(HEADS, NP, HID)
    h2, aT2 = _l2_matmul(out1, b1, W2p, U2)
    out2 = _AGG_CLS(h2, aT2, wpack, counts).reshape(HEADS, NP, CP2)
    res = _final(out2, b2p)
    return res[:N, :N_CLASSES]
